# baseline (speedup 1.0000x reference)
import functools

import jax
import jax.numpy as jnp
from jax import lax
from jax.experimental import pallas as pl
from jax.experimental.pallas import tpu as pltpu
from jax.experimental.pallas import tpu_sc as plsc

N = 10000
D = 128
NC = 2
NS = 16
NT = NC * NS
CH = 128
CPT = 80
EPT = CPT * CH
EP = NT * EPT
NPAD = 10080
RPS = 632

_mesh = plsc.VectorSubcoreMesh(core_axis_name="c", subcore_axis_name="s")
_sc_params = pltpu.CompilerParams(needs_layout_passes=False)


@functools.partial(
    pl.kernel,
    mesh=_mesh,
    out_type=jax.ShapeDtypeStruct((NT, NPAD), jnp.float32),
    compiler_params=_sc_params,
    scratch_types=[
        pltpu.VMEM((EPT,), jnp.int32),
        pltpu.VMEM((NPAD,), jnp.float32),
    ],
)
def _deg_kernel(dst_hbm, out_hbm, dst_v, hist_v):
    c = lax.axis_index("c")
    s = lax.axis_index("s")
    t = c * NS + s
    pltpu.sync_copy(dst_hbm.at[t], dst_v)

    def zero_body(i, carry):
        hist_v[pl.ds(i * 16, 16)] = jnp.zeros((16,), jnp.float32)
        return carry

    lax.fori_loop(0, NPAD // 16, zero_body, 0)

    ones = jnp.ones((16,), jnp.float32)

    def body(i, carry):
        idx = dst_v[pl.ds(i * 16, 16)]
        plsc.addupdate_scatter(hist_v, [idx], ones)
        return carry

    lax.fori_loop(0, EPT // 16, body, 0)
    pltpu.sync_copy(hist_v, out_hbm.at[t])


@functools.partial(
    pl.kernel,
    mesh=_mesh,
    out_type=jax.ShapeDtypeStruct((NC, NPAD, D), jnp.float32),
    compiler_params=_sc_params,
    scratch_types=[
        pltpu.VMEM((CH, D), jnp.float32),
        pltpu.VMEM((CH, D), jnp.float32),
        pltpu.VMEM((CH, D), jnp.float32),
        pltpu.VMEM((2, CH), jnp.int32),
        pltpu.VMEM((2, CH), jnp.int32),
        pltpu.VMEM((2, CH), jnp.int32),
        pltpu.VMEM((2, CH), jnp.int32),
        pltpu.VMEM_SHARED((NPAD, D), jnp.float32),
        pltpu.SemaphoreType.DMA,
        pltpu.SemaphoreType.DMA,
        pltpu.SemaphoreType.DMA,
        pltpu.SemaphoreType.DMA,
        pltpu.SemaphoreType.DMA,
        pltpu.SemaphoreType.DMA,
        pltpu.SemaphoreType.DMA,
        pltpu.SemaphoreType.DMA,
        pltpu.SemaphoreType.DMA,
        pltpu.SemaphoreType.DMA,
    ],
)
def _edge_kernel(
    h_hbm, eidx_hbm, out_hbm,
    f0, f1, f2, i0, i1, i2, i3, acc,
    gs0, gs1, gs2, ss0, ss1, ss2, is0, is1, is2, is3,
):
    c = lax.axis_index("c")
    s = lax.axis_index("s")
    t = c * NS + s
    F = [f0, f1, f2]
    I = [i0, i1, i2, i3]
    gsem = [gs0, gs1, gs2]
    ssem = [ss0, ss1, ss2]
    isem = [is0, is1, is2, is3]

    def istart(g, sl):
        pltpu.async_copy(eidx_hbm.at[t, g], I[sl], isem[sl])

    def iwait(g, sl):
        pltpu.make_async_copy(eidx_hbm.at[t, g], I[sl], isem[sl]).wait()

    def gstart(b3, sl):
        pltpu.async_copy(h_hbm.at[I[sl].at[0]], F[b3], gsem[b3])

    def gwait(b3, sl):
        pltpu.make_async_copy(h_hbm.at[I[sl].at[0]], F[b3], gsem[b3]).wait()

    def sstart(b3, sl):
        pltpu.async_copy(F[b3], acc.at[I[sl].at[1]], ssem[b3], add=True)

    def swait(b3, sl):
        pltpu.make_async_copy(F[b3], acc.at[I[sl].at[1]], ssem[b3]).wait()

    istart(0, 0)
    istart(1, 1)
    istart(2, 2)
    iwait(0, 0)
    gstart(0, 0)
    iwait(1, 1)
    gstart(1, 1)

    def zbody(i, carry):
        r = i // 8
        j = i % 8
        f2[r, pl.ds(j * 16, 16)] = jnp.zeros((16,), jnp.float32)
        return carry

    lax.fori_loop(0, CH * 8, zbody, 0)

    def zcopy(k, carry):
        pltpu.sync_copy(f2, acc.at[pl.ds(s * RPS + k * CH, CH)])
        return carry

    lax.fori_loop(0, 4, zcopy, 0)

    @pl.when(s < NS - 1)
    def _():
        pltpu.sync_copy(
            f2.at[pl.ds(0, RPS - 4 * CH)],
            acc.at[pl.ds(s * RPS + 4 * CH, RPS - 4 * CH)],
        )

    @pl.when(s == NS - 1)
    def _():
        pltpu.sync_copy(
            f2.at[pl.ds(0, NPAD - 15 * RPS - 4 * CH)],
            acc.at[pl.ds(s * RPS + 4 * CH, NPAD - 15 * RPS - 4 * CH)],
        )

    plsc.subcore_barrier()

    nk = (CPT - 8) // 12

    def step(k, carry):
        for b in range(12):
            g = 12 * k + b
            b3 = b % 3
            sl = b % 4
            gwait(b3, sl)
            sstart(b3, sl)
            if b == 0:
                @pl.when(k > 0)
                def _():
                    swait(2, 3)
            else:
                swait((b - 1) % 3, (b - 1) % 4)
            istart(g + 3, (b + 3) % 4)
            iwait(g + 2, (b + 2) % 4)
            gstart((b + 2) % 3, (b + 2) % 4)
        return carry

    lax.fori_loop(0, nk, step, 0)
    for g in range(CPT - 8, CPT):
        gwait(g % 3, g % 4)
        sstart(g % 3, g % 4)
        swait((g - 1) % 3, (g - 1) % 4)
        if g + 3 < CPT:
            istart(g + 3, (g + 3) % 4)
        if g + 2 < CPT:
            iwait(g + 2, (g + 2) % 4)
            gstart((g + 2) % 3, (g + 2) % 4)
    swait((CPT - 1) % 3, (CPT - 1) % 4)
    plsc.subcore_barrier()

    def ocopy(k, carry):
        pltpu.sync_copy(
            acc.at[pl.ds(s * RPS + k * CH, CH)],
            out_hbm.at[c, pl.ds(s * RPS + k * CH, CH)],
        )
        return carry

    lax.fori_loop(0, 4, ocopy, 0)

    @pl.when(s < NS - 1)
    def _():
        pltpu.sync_copy(
            acc.at[pl.ds(s * RPS + 4 * CH, RPS - 4 * CH)],
            out_hbm.at[c, pl.ds(s * RPS + 4 * CH, RPS - 4 * CH)],
        )

    @pl.when(s == NS - 1)
    def _():
        pltpu.sync_copy(
            acc.at[pl.ds(s * RPS + 4 * CH, NPAD - 15 * RPS - 4 * CH)],
            out_hbm.at[c, pl.ds(s * RPS + 4 * CH, NPAD - 15 * RPS - 4 * CH)],
        )


def _m0_body(x_ref, w_ref, m_ref):
    m_ref[...] = jnp.dot(x_ref[...], w_ref[...], preferred_element_type=jnp.float32)


def _m0(x, w):
    return pl.pallas_call(
        _m0_body,
        out_shape=jax.ShapeDtypeStruct((N, D), jnp.float32),
    )(x, w)


def _scale_body(hists_ref, m_ref, h_ref, dis_ref):
    deg = jnp.sum(hists_ref[:, :N], axis=0) + 1.0
    dis = lax.rsqrt(deg)[:, None]
    dis_ref[...] = dis
    h_ref[...] = m_ref[...] * dis


def _scale(hists, m):
    return pl.pallas_call(
        _scale_body,
        out_shape=(
            jax.ShapeDtypeStruct((N, D), jnp.float32),
            jax.ShapeDtypeStruct((N, 1), jnp.float32),
        ),
    )(hists, m)


def _fuse_body(p_ref, h_ref, dis_ref, b_ref, w_ref, o_ref):
    dis = dis_ref[...]
    a = p_ref[0, :N] + p_ref[1, :N] + h_ref[...]
    x = jnp.maximum(a * dis + b_ref[...], 0.0)
    o_ref[...] = jnp.dot(x * dis, w_ref[...], preferred_element_type=jnp.float32)


def _fuse(p, h, dis, b, w):
    return pl.pallas_call(
        _fuse_body,
        out_shape=jax.ShapeDtypeStruct((N, D), jnp.float32),
    )(p, h, dis, b, w)


def _final_body(p_ref, h_ref, dis_ref, b_ref, o_ref):
    a = p_ref[0, :N] + p_ref[1, :N] + h_ref[...]
    o_ref[...] = a * dis_ref[...] + b_ref[...]


def _final(p, h, dis, b):
    return pl.pallas_call(
        _final_body,
        out_shape=jax.ShapeDtypeStruct((N, D), jnp.float32),
    )(p, h, dis, b)


def kernel(x, edge_index, W0, b0, W1, b1, W2, b2, W3, b3):
    src = edge_index[0].astype(jnp.int32)
    dst = edge_index[1].astype(jnp.int32)
    e = src.shape[0]
    pad_n = EP - e
    ar = jnp.arange(pad_n, dtype=jnp.int32)
    src_p = jnp.concatenate([src, ar % N]).reshape(NT, CPT, CH)
    dst_p = jnp.concatenate([dst, N + ar % (NPAD - N)]).reshape(NT, CPT, CH)
    eidx = jnp.stack([src_p, dst_p], axis=2)
    dst_flat = dst_p.reshape(NT, EPT)

    m = _m0(x, W0)
    hists = _deg_kernel(dst_flat)
    h, dis = _scale(hists, m)
    b_prev = [b0, b1, b2]
    w_next = [W1, W2, W3]
    for i in range(3):
        p = _edge_kernel(h, eidx)
        h = _fuse(p, h, dis, b_prev[i].reshape(1, D), w_next[i])
    p = _edge_kernel(h, eidx)
    return _final(p, h, dis, b3.reshape(1, D))
